# Initial kernel scaffold; baseline (speedup 1.0000x reference)
import jax
import jax.numpy as jnp
from jax.experimental import pallas as pl


def kernel(embedding, hidden_states, output_positions, temperatures, top_ps, top_ks):
    raise NotImplementedError("write your pallas kernel here")


# speedup vs baseline: 10.2113x; 10.2113x over previous
import functools

import jax
import jax.numpy as jnp
from jax import lax
from jax.experimental import pallas as pl
from jax.experimental.pallas import tpu as pltpu
from jax.experimental.pallas import tpu_sc as plsc

_V = 100000
_D = 4096
_B = 128
_SOFTCAP = 30.0
_K = 64
_CAP = 128
_TV = 512
_RB = 8


def _matmul_body(hs_ref, emb_ref, temp_ref, out_ref):
    h = hs_ref[...]
    e = emb_ref[...]
    acc = lax.dot_general(h, e, (((1,), (1,)), ((), ())),
                          preferred_element_type=jnp.float32)
    l = jnp.tanh(acc / _SOFTCAP) * _SOFTCAP
    out_ref[...] = l / temp_ref[...]


def _logits_kernel(hs, emb, temps2d):
    nt = pl.cdiv(_V, _TV)
    return pl.pallas_call(
        _matmul_body,
        grid=(nt,),
        in_specs=[
            pl.BlockSpec((_B, _D), lambda i: (0, 0)),
            pl.BlockSpec((_TV, _D), lambda i: (i, 0)),
            pl.BlockSpec((_B, 1), lambda i: (0, 0)),
        ],
        out_specs=pl.BlockSpec((_B, _TV), lambda i: (0, i)),
        out_shape=jax.ShapeDtypeStruct((_B, _V), jnp.float32),
        compiler_params=pltpu.CompilerParams(
            dimension_semantics=("arbitrary",)),
    )(hs, emb, temps2d)


def _select_body(l_ref, t_ref, m_ref, z_ref):
    l = l_ref[...]
    m = jnp.max(l, axis=1, keepdims=True)
    z = jnp.sum(jnp.exp(l - m), axis=1, keepdims=True)
    s = lax.bitcast_convert_type(l, jnp.int32)
    key = jnp.where(s < 0, s ^ jnp.int32(0x7FFFFFFF), s)

    cnt0 = jnp.sum((key >= 0).astype(jnp.int32), axis=1, keepdims=True)
    p0 = jnp.where(cnt0 >= _K, jnp.int32(0), jnp.int32(-2147483648))

    def step(i, p):
        cand = p | lax.shift_left(jnp.int32(1), jnp.int32(30) - i)
        cnt = jnp.sum((key >= cand).astype(jnp.int32), axis=1, keepdims=True)
        return jnp.where(cnt >= _K, cand, p)

    p = lax.fori_loop(0, 31, step, p0)
    tb = jnp.where(p < 0, p ^ jnp.int32(0x7FFFFFFF), p)
    t = lax.bitcast_convert_type(tb, jnp.float32)
    t_ref[...] = jnp.broadcast_to(t, (_RB, 128))
    m_ref[...] = jnp.broadcast_to(m, (_RB, 128))
    z_ref[...] = jnp.broadcast_to(z, (_RB, 128))


def _select_kernel(logits):
    g = _B // _RB
    shp = jax.ShapeDtypeStruct((_B, 128), jnp.float32)
    return pl.pallas_call(
        _select_body,
        grid=(g,),
        in_specs=[pl.BlockSpec((_RB, _V), lambda i: (i, 0))],
        out_specs=[pl.BlockSpec((_RB, 128), lambda i: (i, 0))] * 3,
        out_shape=[shp, shp, shp],
        compiler_params=pltpu.CompilerParams(
            dimension_semantics=("arbitrary",)),
    )(logits)


def _sc_sample(logits, params, gflat):
    info = plsc.get_sparse_core_info()
    nc, ns = info.num_cores, info.num_subcores
    nw = nc * ns
    rpw = _B // nw
    mesh = plsc.VectorSubcoreMesh(core_axis_name="c", subcore_axis_name="s")

    @functools.partial(
        pl.kernel,
        out_type=jax.ShapeDtypeStruct((nw, 16), jnp.int32),
        mesh=mesh,
        scratch_types=[
            pltpu.VMEM((_V,), jnp.float32),
            pltpu.VMEM((80,), jnp.float32),
            pltpu.VMEM((_CAP,), jnp.float32),
            pltpu.VMEM((_CAP,), jnp.int32),
            pltpu.VMEM((_CAP,), jnp.float32),
            pltpu.VMEM((_CAP,), jnp.int32),
            pltpu.VMEM((_CAP,), jnp.float32),
            pltpu.VMEM((16,), jnp.int32),
            pltpu.SemaphoreType.DMA,
        ],
        compiler_params=pltpu.CompilerParams(needs_layout_passes=False),
    )
    def k(l_hbm, par_hbm, g_hbm, out_hbm,
          row_v, par_v, cval, cidx, pval, gidx, gval, tok_v, sem):
        cid = lax.axis_index("c")
        sid = lax.axis_index("s")
        wid = sid * nc + cid
        lane = lax.iota(jnp.int32, 16)
        neginf = jnp.float32(-1e30)
        nchunks = _CAP // 16

        def do_row(kk, tok_acc):
            r = wid * rpw + kk
            pltpu.sync_copy(l_hbm.at[r], row_v)
            pltpu.sync_copy(par_hbm.at[r], par_v)

            thr_v = par_v[pl.ds(0, 16)]
            m_v = par_v[pl.ds(16, 16)]
            z_v = par_v[pl.ds(32, 16)]
            top_pv = par_v[pl.ds(48, 16)]
            top_kv = par_v[pl.ds(64, 16)].astype(jnp.int32)

            def initb(j, c):
                cval[pl.ds(j * 16, 16)] = jnp.broadcast_to(neginf, (16,))
                cidx[pl.ds(j * 16, 16)] = jnp.broadcast_to(jnp.int32(0), (16,))
                return c

            lax.fori_loop(0, nchunks, initb, jnp.int32(0))

            def comp(i, cnt):
                v = row_v[pl.ds(i * 16, 16)]
                msk = v >= thr_v
                base = jnp.minimum(cnt, jnp.broadcast_to(
                    jnp.int32(_CAP - 16), (16,)))[0]
                plsc.store_compressed(cval.at[pl.ds(base, 16)], v, mask=msk)
                plsc.store_compressed(cidx.at[pl.ds(base, 16)],
                                      lane + i * 16, mask=msk)
                return cnt + plsc.all_reduce_population_count(msk)

            lax.fori_loop(0, _V // 16, comp,
                          jnp.broadcast_to(jnp.int32(0), (16,)))

            roff = jnp.broadcast_to(r * jnp.int32(_V), (16,))

            def prep(j, c):
                cv = cval[pl.ds(j * 16, 16)]
                pval[pl.ds(j * 16, 16)] = jnp.exp(cv - m_v) / z_v
                gidx[pl.ds(j * 16, 16)] = cidx[pl.ds(j * 16, 16)] + roff
                return c

            lax.fori_loop(0, nchunks, prep, jnp.int32(0))

            pltpu.async_copy(g_hbm.at[gidx], gval, sem).wait()

            zero_i = jnp.broadcast_to(jnp.int32(0), (16,))
            zero_f = jnp.broadcast_to(jnp.float32(0.0), (16,))
            ninf_v = jnp.broadcast_to(neginf, (16,))

            def score_chunk(t, carry):
                best_v, best_i = carry
                av = cval[pl.ds(t * 16, 16)]
                ai = cidx[pl.ds(t * 16, 16)]

                def jbody(j, rc):
                    rnk, cum = rc
                    js = jnp.broadcast_to(j, (16,))
                    bvs = plsc.load_gather(cval, [js])
                    bis = plsc.load_gather(cidx, [js])
                    bps = plsc.load_gather(pval, [js])
                    gt = (bvs > av) | ((bvs == av) & (bis < ai))
                    return (rnk + gt.astype(jnp.int32),
                            cum + jnp.where(gt, bps, zero_f))

                rnk, cum = lax.fori_loop(0, _CAP, jbody, (zero_i, zero_f))
                kept = (rnk < top_kv) & (cum <= top_pv)
                gv = gval[pl.ds(t * 16, 16)]
                score = jnp.where(kept, av + gv, ninf_v)
                upd = score > best_v
                return (jnp.where(upd, score, best_v),
                        jnp.where(upd, ai, best_i))

            best_v, best_i = lax.fori_loop(0, nchunks, score_chunk,
                                           (ninf_v, zero_i))
            _, srt_i = plsc.sort_key_val(best_v, best_i, descending=True)
            tok = srt_i[0]
            return jnp.where(lane == kk, jnp.broadcast_to(tok, (16,)), tok_acc)

        tok_acc = lax.fori_loop(0, rpw, do_row,
                                jnp.broadcast_to(jnp.int32(0), (16,)))
        tok_v[...] = tok_acc
        pltpu.sync_copy(tok_v, out_hbm.at[wid])

    return k(logits, params, gflat)


def kernel(embedding, hidden_states, output_positions, temperatures,
           top_ps, top_ks):
    hs = jnp.take(hidden_states, output_positions, axis=1)
    hs = jnp.squeeze(hs, axis=1)
    logits = _logits_kernel(hs, embedding, temperatures[:, None])
    t_b, m_b, z_b = _select_kernel(logits)
    fields = jnp.stack([t_b[:, 0], m_b[:, 0], z_b[:, 0], top_ps,
                        top_ks.astype(jnp.float32)], axis=1)
    params = jnp.repeat(fields[:, :, None], 16, axis=2).reshape(_B, 80)
    g = jax.random.gumbel(jax.random.key(42), (_B, _V), jnp.float32)
    toks = _sc_sample(logits, params, g.reshape(-1))
    nw = toks.shape[0]
    next_ids = toks[:, : _B // nw].reshape(_B)
    return next_ids, logits
